# baseline (speedup 1.0000x reference)
import jax
import jax.numpy as jnp
import numpy as np
from jax import lax
from jax.experimental import pallas as pl
from jax.experimental.pallas import tpu as pltpu
from jax.experimental.pallas import tpu_sc as plsc

D = 64
_TRIL = D * (D + 1) // 2
_XW = D + _TRIL
_NC, _NS = 2, 16
_NW = _NC * _NS
_R = 4
_CHUNKS = _TRIL // 16


def _sc_body(x_hbm, ti_hbm, tj_hbm, loc_hbm, out_hbm,
             x_v0, x_v1, ti_v, tj_v, loc_v0, loc_v1, out_v0, out_v1,
             in_s0, in_s1, out_s0, out_s1, loc_s0, loc_s1):
    c = lax.axis_index("c")
    s = lax.axis_index("s")
    wid = s * _NC + c
    rows_per_w = x_hbm.shape[0] // _NW
    groups = rows_per_w // _R
    base0 = wid * rows_per_w

    x_v = [x_v0, x_v1]
    loc_v = [loc_v0, loc_v1]
    out_v = [out_v0, out_v1]
    in_s = [in_s0, in_s1]
    out_s = [out_s0, out_s1]
    loc_s = [loc_s0, loc_s1]

    pltpu.sync_copy(ti_hbm, ti_v)
    pltpu.sync_copy(tj_hbm, tj_v)

    zeros16 = jnp.zeros((16,), jnp.float32)

    for r in range(_R):
        def zero_body(i, carry):
            for cc in range(D // 16):
                out_v0[r, i, pl.ds(cc * 16, 16)] = zeros16
                out_v1[r, i, pl.ds(cc * 16, 16)] = zeros16
            return carry

        lax.fori_loop(0, D, zero_body, 0)

    def in_copy(g, b):
        base = base0 + g * _R
        return pltpu.make_async_copy(
            x_hbm.at[pl.ds(base, _R)], x_v[b], in_s[b])

    def out_copy(g, b):
        base = base0 + g * _R
        return pltpu.make_async_copy(
            out_v[b], out_hbm.at[pl.ds(base, _R)], out_s[b])

    def loc_copy(g, b):
        base = base0 + g * _R
        return pltpu.make_async_copy(
            loc_v[b], loc_hbm.at[pl.ds(base, _R)], loc_s[b])

    in_copy(0, 0).start()
    in_copy(1, 1).start()

    def compute(g, b):
        xb, ob, lb = x_v[b], out_v[b], loc_v[b]

        def chunk_body(k, carry2):
            di = ti_v[pl.ds(k * 16, 16)]
            dj = tj_v[pl.ds(k * 16, 16)]
            diag = di == dj
            for rr in range(_R):
                v = xb[rr, pl.ds(D + k * 16, 16)]
                v = jnp.where(diag, jnp.exp(v), v)
                plsc.store_scatter(
                    ob, [jnp.full((16,), rr, jnp.int32), di, dj], v)
            return carry2

        lax.fori_loop(0, _CHUNKS, chunk_body, 0)

        for rr in range(_R):
            for cc in range(4):
                lb[rr, pl.ds(cc * 16, 16)] = xb[rr, pl.ds(cc * 16, 16)]

    def group_body(gg, carry):
        for b in range(2):
            g = gg * 2 + b
            in_copy(g, b).wait()

            @pl.when(g >= 2)
            def _():
                out_copy(g - 2, b).wait()
                loc_copy(g - 2, b).wait()

            compute(g, b)
            out_copy(g, b).start()
            loc_copy(g, b).start()

            @pl.when(g + 2 < groups)
            def _():
                in_copy(g + 2, b).start()
        return carry

    lax.fori_loop(0, groups // 2, group_body, 0)

    for b in range(2):
        g = groups - 2 + b
        out_copy(g, b).wait()
        loc_copy(g, b).wait()


def kernel(x):
    B = x.shape[0]
    ti, tj = np.tril_indices(D)
    ti_a = jnp.asarray(ti.astype(np.int32))
    tj_a = jnp.asarray(tj.astype(np.int32))

    mesh = plsc.VectorSubcoreMesh(
        core_axis_name="c", subcore_axis_name="s",
        num_cores=_NC, num_subcores=_NS)
    run = pl.kernel(
        _sc_body,
        out_type=[
            jax.ShapeDtypeStruct((B, D), jnp.float32),
            jax.ShapeDtypeStruct((B, D, D), jnp.float32),
        ],
        mesh=mesh,
        scratch_types=[
            pltpu.VMEM((_R, _XW), jnp.float32),
            pltpu.VMEM((_R, _XW), jnp.float32),
            pltpu.VMEM((_TRIL,), jnp.int32),
            pltpu.VMEM((_TRIL,), jnp.int32),
            pltpu.VMEM((_R, D), jnp.float32),
            pltpu.VMEM((_R, D), jnp.float32),
            pltpu.VMEM((_R, D, D), jnp.float32),
            pltpu.VMEM((_R, D, D), jnp.float32),
            pltpu.SemaphoreType.DMA,
            pltpu.SemaphoreType.DMA,
            pltpu.SemaphoreType.DMA,
            pltpu.SemaphoreType.DMA,
            pltpu.SemaphoreType.DMA,
            pltpu.SemaphoreType.DMA,
        ],
        compiler_params=pltpu.CompilerParams(needs_layout_passes=False),
    )
    loc, out = run(x, ti_a, tj_a)
    return loc, out
